# baseline (speedup 1.0000x reference)
import jax
import jax.numpy as jnp
from jax import lax
from jax.experimental import pallas as pl
from jax.experimental.pallas import tpu as pltpu
from jax.experimental.pallas import tpu_sc as plsc

_BATCH = 16384
_EMBED_DIM = 64
_VOCAB = 100001
_NUM_CORES = 2
_NUM_SUBCORES = 16
_NUM_WORKERS = _NUM_CORES * _NUM_SUBCORES
_DIMS_PER_W = _EMBED_DIM // _NUM_WORKERS
_CHUNK = 4096
_NUM_CHUNKS = _BATCH // _CHUNK
_LANES = 16
_UNROLL = 16


def _gather_body(idx_hbm, tblt_hbm, outt_hbm, row_v, idx_v, out_v, idx_sp, rsem, osem):
    sid = lax.axis_index("s")
    wid = sid * _NUM_CORES + lax.axis_index("c")
    row_dma = pltpu.async_copy(tblt_hbm.at[wid * _DIMS_PER_W], row_v, rsem)

    @pl.when(sid == 0)
    def _():
        pltpu.sync_copy(idx_hbm, idx_sp)

    plsc.subcore_barrier()
    pltpu.sync_copy(idx_sp, idx_v)
    pending = []
    for j in range(_DIMS_PER_W):
        d = wid * _DIMS_PER_W + j
        row_dma.wait()
        for c in range(_NUM_CHUNKS):
            buf = (j * _NUM_CHUNKS + c) % 2
            if len(pending) >= 2:
                pending.pop(0).wait()

            @plsc.parallel_loop(0, _CHUNK, step=_LANES, unroll=_UNROLL)
            def _(o, _c=c, _buf=buf):
                iv = idx_v[pl.ds(_c * _CHUNK + o, _LANES)]
                out_v[_buf, pl.ds(o, _LANES)] = plsc.load_gather(row_v, [iv])

            pending.append(
                pltpu.async_copy(
                    out_v.at[buf],
                    outt_hbm.at[d, pl.ds(c * _CHUNK, _CHUNK)],
                    osem,
                )
            )
        if j + 1 < _DIMS_PER_W:
            row_dma = pltpu.async_copy(tblt_hbm.at[d + 1], row_v, rsem)
    for p in pending:
        p.wait()


@jax.jit
def _gather(indices, table):
    mesh = plsc.VectorSubcoreMesh(
        core_axis_name="c",
        subcore_axis_name="s",
        num_cores=_NUM_CORES,
        num_subcores=_NUM_SUBCORES,
    )
    out_t = pl.kernel(
        _gather_body,
        out_type=jax.ShapeDtypeStruct((_EMBED_DIM, _BATCH), jnp.float32),
        mesh=mesh,
        scratch_types=[
            pltpu.VMEM((_VOCAB,), jnp.float32),
            pltpu.VMEM((_BATCH,), jnp.int32),
            pltpu.VMEM((2, _CHUNK), jnp.float32),
            pltpu.VMEM_SHARED((_BATCH,), jnp.int32),
            pltpu.SemaphoreType.DMA,
            pltpu.SemaphoreType.DMA,
        ],
        compiler_params=pltpu.CompilerParams(needs_layout_passes=False),
    )(indices, table.T)
    return out_t.T


def kernel(indices, table):
    return _gather(indices.astype(jnp.int32), table)


# speedup vs baseline: 1.0130x; 1.0130x over previous
import jax
import jax.numpy as jnp
from jax import lax
from jax.experimental import pallas as pl
from jax.experimental.pallas import tpu as pltpu
from jax.experimental.pallas import tpu_sc as plsc

_BATCH = 16384
_EMBED_DIM = 64
_VOCAB = 100001
_NUM_CORES = 2
_NUM_SUBCORES = 16
_NUM_WORKERS = _NUM_CORES * _NUM_SUBCORES
_DIMS_PER_W = _EMBED_DIM // _NUM_WORKERS
_CHUNK = 4096
_NUM_CHUNKS = _BATCH // _CHUNK
_LANES = 16
_UNROLL = 8


def _gather_body(idx_hbm, tblt_hbm, outt_hbm, row_v, idx_v, out_v, idx_sp, rsem, osem):
    sid = lax.axis_index("s")
    wid = sid * _NUM_CORES + lax.axis_index("c")
    row_dma = pltpu.async_copy(tblt_hbm.at[wid * _DIMS_PER_W], row_v, rsem)

    @pl.when(sid == 0)
    def _():
        pltpu.sync_copy(idx_hbm, idx_sp)

    plsc.subcore_barrier()
    pltpu.sync_copy(idx_sp, idx_v)
    pending = []
    for j in range(_DIMS_PER_W):
        d = wid * _DIMS_PER_W + j
        row_dma.wait()
        for c in range(_NUM_CHUNKS):
            buf = (j * _NUM_CHUNKS + c) % 2
            if len(pending) >= 2:
                pending.pop(0).wait()

            @plsc.parallel_loop(0, _CHUNK, step=_LANES, unroll=_UNROLL)
            def _(o, _c=c, _buf=buf):
                iv = idx_v[pl.ds(_c * _CHUNK + o, _LANES)]
                out_v[_buf, pl.ds(o, _LANES)] = plsc.load_gather(row_v, [iv])

            pending.append(
                pltpu.async_copy(
                    out_v.at[buf],
                    outt_hbm.at[d, pl.ds(c * _CHUNK, _CHUNK)],
                    osem,
                )
            )
        if j + 1 < _DIMS_PER_W:
            row_dma = pltpu.async_copy(tblt_hbm.at[d + 1], row_v, rsem)
    for p in pending:
        p.wait()


@jax.jit
def _gather(indices, table):
    mesh = plsc.VectorSubcoreMesh(
        core_axis_name="c",
        subcore_axis_name="s",
        num_cores=_NUM_CORES,
        num_subcores=_NUM_SUBCORES,
    )
    out_t = pl.kernel(
        _gather_body,
        out_type=jax.ShapeDtypeStruct((_EMBED_DIM, _BATCH), jnp.float32),
        mesh=mesh,
        scratch_types=[
            pltpu.VMEM((_VOCAB,), jnp.float32),
            pltpu.VMEM((_BATCH,), jnp.int32),
            pltpu.VMEM((2, _CHUNK), jnp.float32),
            pltpu.VMEM_SHARED((_BATCH,), jnp.int32),
            pltpu.SemaphoreType.DMA,
            pltpu.SemaphoreType.DMA,
        ],
        compiler_params=pltpu.CompilerParams(needs_layout_passes=False),
    )(indices, table.T)
    return out_t.T


def kernel(indices, table):
    return _gather(indices.astype(jnp.int32), table)
